# Initial kernel scaffold; baseline (speedup 1.0000x reference)
import jax
import jax.numpy as jnp
from jax.experimental import pallas as pl


def kernel(y_pred, y_true, P_padded, params):
    raise NotImplementedError("write your pallas kernel here")


# speedup vs baseline: 226.9465x; 226.9465x over previous
import functools

import jax
import jax.numpy as jnp
import numpy as np
from jax.experimental import pallas as pl
from jax.experimental.pallas import tpu as pltpu

B = 16384
S = 20
NBLK = 16
BR = 1024
LAMBDA_VIOLATION = np.float32(0.5)
EPS = np.float32(1e-09)
U32 = jnp.uint32

VALID_COMBOS = [
    (n, k, m)
    for n in range(4)
    for k in range(4)
    for m in range(4)
    if (m + 1) <= n and k > 0 and (n - k) >= 0
]
VALID_IDS = [n * 16 + k * 4 + m for (n, k, m) in VALID_COMBOS]

_ROTS = ((13, 15, 26, 6), (17, 29, 16, 24))


def _threefry2x32(k0, k1, x0, x1):
    k2 = k0 ^ k1 ^ U32(0x1BD11BDA)
    ks = (k0, k1, k2)
    x0 = x0 + k0
    x1 = x1 + k1
    for i in range(5):
        for r in _ROTS[i % 2]:
            x0 = x0 + x1
            x1 = (x1 << U32(r)) | (x1 >> U32(32 - r))
            x1 = x1 ^ x0
        x0 = x0 + ks[(i + 1) % 3]
        x1 = x1 + ks[(i + 2) % 3] + U32(i + 1)
    return x0, x1


_GILES_LO = [2.81022636e-08, 3.43273939e-07, -3.5233877e-06, -4.39150654e-06,
             0.00021858087, -0.00125372503, -0.00417768164, 0.246640727,
             1.50140941]
_GILES_HI = [-0.000200214257, 0.000100950558, 0.00134934322, -0.00367342844,
             0.00573950773, -0.0076224613, 0.00943887047, 1.00167406,
             2.83297682]


def _erf_inv(x):
    w = -jnp.log1p(-x * x)
    w_lo = w - jnp.float32(5 / 2)
    w_hi = jnp.sqrt(w) - jnp.float32(3.0)
    p_lo = jnp.float32(_GILES_LO[0])
    p_hi = jnp.float32(_GILES_HI[0])
    for c_lo, c_hi in zip(_GILES_LO[1:], _GILES_HI[1:]):
        p_lo = jnp.float32(c_lo) + p_lo * w_lo
        p_hi = jnp.float32(c_hi) + p_hi * w_hi
    return jnp.where(w < jnp.float32(5.0), p_lo, p_hi) * x


def _normal_from_bits(bits):
    float_bits = (bits >> U32(9)) | U32(0x3F800000)
    floats = jax.lax.bitcast_convert_type(float_bits, jnp.float32) - jnp.float32(1.0)
    lo = jnp.float32(np.nextafter(np.float32(-1.0), np.float32(0.0)))
    hi = jnp.float32(1.0)
    u = jnp.maximum(lo, floats * (hi - lo) + lo)
    return jnp.float32(np.sqrt(np.float32(2.0))) * jax.lax.erf_inv(u)


def _sort6_desc(v):
    v = list(v)
    for rnd in range(6):
        pairs = ((0, 1), (2, 3), (4, 5)) if rnd % 2 == 0 else ((1, 2), (3, 4))
        for i, j in pairs:
            hi = jnp.maximum(v[i], v[j])
            lo = jnp.minimum(v[i], v[j])
            v[i], v[j] = hi, lo
    return v


def _loss_kernel(yp_ref, yt_ref, p_ref, pa_ref, out_total, out_logmse,
                 out_viol, combo_s, rank_s):
    izero = jnp.int32(0)

    def init_body(i, _):
        c = pa_ref[0, i] * 16 + pa_ref[1, i] * 4 + pa_ref[2, i]
        combo_s[i] = c
        rank_s[i] = jnp.zeros((8, 128), jnp.float32)
        return _

    jax.lax.fori_loop(0, NBLK, init_body, izero)

    r8 = jax.lax.broadcasted_iota(jnp.int32, (8, 8), 0)
    c8 = jax.lax.broadcasted_iota(jnp.int32, (8, 8), 1)
    ltri8 = (r8 > c8).astype(jnp.float32)
    ra = jax.lax.broadcasted_iota(jnp.int32, (128, 128), 0)
    cb = jax.lax.broadcasted_iota(jnp.int32, (128, 128), 1)
    tin128 = (ra <= cb).astype(jnp.float32)

    counts = [None] * 64
    for c in range(64):
        cc = jnp.int32(c)
        if c in VALID_IDS:
            def rank_body(i, counter, cc=cc):
                mask = (combo_s[i] == cc).astype(jnp.float32)
                lane_cs = jax.lax.dot_general(
                    mask, tin128, (((1,), (0,)), ((), ())),
                    preferred_element_type=jnp.float32)
                sub_tot = lane_cs[:, 127:128]
                excl_sub = jax.lax.dot_general(
                    ltri8, sub_tot, (((1,), (0,)), ((), ())),
                    preferred_element_type=jnp.float32)
                prefix_excl = lane_cs - mask + excl_sub
                rank_s[i] = rank_s[i] + mask * (prefix_excl + counter)
                return counter + jnp.sum(mask)

            counts[c] = jax.lax.fori_loop(0, NBLK, rank_body, jnp.float32(0.0))
        else:
            def cnt_body(i, counter, cc=cc):
                return counter + jnp.sum((combo_s[i] == cc).astype(jnp.float32))

            counts[c] = jax.lax.fori_loop(0, NBLK, cnt_body, jnp.float32(0.0))

    present = [(counts[c] > 0.0).astype(jnp.int32) for c in range(64)]
    pref = []
    acc = izero
    for c in range(64):
        acc = acc + present[c]
        pref.append(acc)
    combo_keys = {}
    for c in VALID_IDS:
        idx = (pref[c] - 1).astype(U32)
        k0c, k1c = _threefry2x32(U32(0), U32(42), U32(0), idx)
        combo_keys[c] = (k0c, k1c)

    sqrt2 = jnp.float32(np.sqrt(np.float32(2.0)))
    inv_ln2 = jnp.float32(1.0 / np.log(2.0))

    def main_body(i, carry):
        acc_pen, acc_log = carry
        c_t = combo_s[i]
        n_t = c_t // 16
        k_t = (c_t // 4) % 4
        m_t = c_t % 4
        valid = (m_t < n_t) & (k_t >= 1) & (k_t <= n_t)

        k0r = jnp.zeros((8, 128), U32)
        k1r = jnp.zeros((8, 128), U32)
        for c in VALID_IDS:
            sel = c_t == c
            k0c, k1c = combo_keys[c]
            k0r = jnp.where(sel, k0c, k0r)
            k1r = jnp.where(sel, k1c, k1r)

        rank_t = rank_s[i].astype(jnp.int32)
        base = (rank_t * (S * k_t)).astype(U32)
        ku = k_t.astype(U32)

        pm = [[None] * 2 for _ in range(2)]
        pmb = [[None] * 2 for _ in range(2)]
        for j in range(2):
            for jj in range(2):
                keep = (j < k_t) & (jj < (n_t - k_t))
                pv = jnp.where(keep, p_ref[j * 3 + jj, i], jnp.float32(0.0))
                pm[j][jj] = pv
                pmb[j][jj] = pv.astype(jnp.bfloat16).astype(jnp.float32)

        k_is1 = k_t == 1
        neg1 = jnp.float32(-1.0)
        max_hm = jnp.full((8, 128), -jnp.inf, jnp.float32)
        for s in range(S):
            a = []
            abf = []
            for j in range(3):
                cnt = base + U32(s) * ku + U32(j)
                o0, o1 = _threefry2x32(k0r, k1r, U32(0), cnt)
                aj = _normal_from_bits(o0 ^ o1)
                aj = jnp.where(j < k_t, aj, jnp.float32(0.0))
                a.append(aj)
                abf.append(aj.astype(jnp.bfloat16).astype(jnp.float32))
            mags = []
            for j in range(3):
                ident = jnp.where(k_is1, a[j], abf[j])
                mags.append(jnp.where(j < k_t, jnp.abs(ident), neg1))
            for jj in range(2):
                e1 = a[0] * pm[0][jj]
                e2 = abf[0] * pmb[0][jj] + abf[1] * pmb[1][jj]
                e = jnp.where(k_is1, e1, e2)
                mags.append(jnp.where(jj < (n_t - k_t), jnp.abs(e), neg1))
            mags.append(neg1 + jnp.zeros((8, 128), jnp.float32))
            t = _sort6_desc(mags)
            t_m = jnp.where(m_t == 0, t[0],
                            jnp.where(m_t == 1, t[1], t[2]))
            hm = t[0] / (t_m + EPS)
            max_hm = jnp.maximum(max_hm, hm)

        yp = yp_ref[i]
        pen = jnp.where(valid, jnp.maximum(max_hm - yp, jnp.float32(0.0)),
                        jnp.float32(0.0))

        ypc = jnp.maximum(yp, EPS)
        ytc = jnp.maximum(yt_ref[i], EPS)
        d = (jnp.log(ytc) - jnp.log(ypc)) * inv_ln2
        return acc_pen + jnp.sum(pen), acc_log + jnp.sum(d * d)

    acc_pen, acc_log = jax.lax.fori_loop(
        0, NBLK, main_body, (jnp.float32(0.0), jnp.float32(0.0)))

    logmse = acc_log / jnp.float32(B)
    violation = acc_pen / jnp.float32(B)
    out_logmse[0, 0] = logmse
    out_viol[0, 0] = violation
    out_total[0, 0] = logmse + LAMBDA_VIOLATION * violation


@functools.partial(jax.jit, static_argnames=("interpret",))
def kernel(y_pred, y_true, P_padded, params, interpret=False):
    yp = y_pred.reshape(NBLK, 8, 128)
    yt = y_true.reshape(NBLK, 8, 128)
    pc = jnp.transpose(P_padded[:, :3, :3], (1, 2, 0)).reshape(9, NBLK, 8, 128)
    pa = jnp.transpose(params.astype(jnp.int32), (1, 0)).reshape(3, NBLK, 8, 128)

    scal = jax.ShapeDtypeStruct((1, 1), jnp.float32)
    out_specs = [pl.BlockSpec(memory_space=pltpu.SMEM)] * 3
    total, logmse, violation = pl.pallas_call(
        _loss_kernel,
        out_shape=[scal, scal, scal],
        out_specs=out_specs,
        scratch_shapes=[
            pltpu.VMEM((NBLK, 8, 128), jnp.int32),
            pltpu.VMEM((NBLK, 8, 128), jnp.float32),
        ],
        interpret=interpret,
    )(yp, yt, pc, pa)
    return (total[0, 0], logmse[0, 0], violation[0, 0])


# speedup vs baseline: 607.8389x; 2.6783x over previous
import functools

import jax
import jax.numpy as jnp
import numpy as np
from jax.experimental import pallas as pl
from jax.experimental.pallas import tpu as pltpu

B = 16384
S = 20
NBLK = 16
BR = 1024
LAMBDA_VIOLATION = np.float32(0.5)
EPS = np.float32(1e-09)
U32 = jnp.uint32

VALID_COMBOS = [
    (n, k, m)
    for n in range(4)
    for k in range(4)
    for m in range(4)
    if (m + 1) <= n and k > 0 and (n - k) >= 0
]
VALID_IDS = [n * 16 + k * 4 + m for (n, k, m) in VALID_COMBOS]

_ROTS = ((13, 15, 26, 6), (17, 29, 16, 24))


def _threefry2x32(k0, k1, x0, x1):
    k2 = k0 ^ k1 ^ U32(0x1BD11BDA)
    ks = (k0, k1, k2)
    x0 = x0 + k0
    x1 = x1 + k1
    for i in range(5):
        for r in _ROTS[i % 2]:
            x0 = x0 + x1
            x1 = (x1 << U32(r)) | (x1 >> U32(32 - r))
            x1 = x1 ^ x0
        x0 = x0 + ks[(i + 1) % 3]
        x1 = x1 + ks[(i + 2) % 3] + U32(i + 1)
    return x0, x1


_GILES_LO = [2.81022636e-08, 3.43273939e-07, -3.5233877e-06, -4.39150654e-06,
             0.00021858087, -0.00125372503, -0.00417768164, 0.246640727,
             1.50140941]
_GILES_HI = [-0.000200214257, 0.000100950558, 0.00134934322, -0.00367342844,
             0.00573950773, -0.0076224613, 0.00943887047, 1.00167406,
             2.83297682]


def _erf_inv(x):
    w = -jnp.log1p(-x * x)
    w_lo = w - jnp.float32(5 / 2)
    w_hi = jnp.sqrt(w) - jnp.float32(3.0)
    p_lo = jnp.float32(_GILES_LO[0])
    p_hi = jnp.float32(_GILES_HI[0])
    for c_lo, c_hi in zip(_GILES_LO[1:], _GILES_HI[1:]):
        p_lo = jnp.float32(c_lo) + p_lo * w_lo
        p_hi = jnp.float32(c_hi) + p_hi * w_hi
    return jnp.where(w < jnp.float32(5.0), p_lo, p_hi) * x


def _normal_from_bits(bits):
    float_bits = (bits >> U32(9)) | U32(0x3F800000)
    floats = jax.lax.bitcast_convert_type(float_bits, jnp.float32) - jnp.float32(1.0)
    lo = jnp.float32(np.nextafter(np.float32(-1.0), np.float32(0.0)))
    hi = jnp.float32(1.0)
    u = jnp.maximum(lo, floats * (hi - lo) + lo)
    return jnp.float32(np.sqrt(np.float32(2.0))) * jax.lax.erf_inv(u)


def _sort6_desc(v):
    v = list(v)
    for rnd in range(6):
        pairs = ((0, 1), (2, 3), (4, 5)) if rnd % 2 == 0 else ((1, 2), (3, 4))
        for i, j in pairs:
            hi = jnp.maximum(v[i], v[j])
            lo = jnp.minimum(v[i], v[j])
            v[i], v[j] = hi, lo
    return v


def _loss_kernel(yp_ref, yt_ref, p_ref, pa_ref, out_total, out_logmse,
                 out_viol, combo_s, rank_s):
    izero = jnp.int32(0)

    def init_body(i, _):
        c = pa_ref[0, i] * 16 + pa_ref[1, i] * 4 + pa_ref[2, i]
        combo_s[i] = c
        rank_s[i] = jnp.zeros((8, 128), jnp.float32)
        return _

    jax.lax.fori_loop(0, NBLK, init_body, izero)

    r8 = jax.lax.broadcasted_iota(jnp.int32, (8, 8), 0)
    c8 = jax.lax.broadcasted_iota(jnp.int32, (8, 8), 1)
    ltri8 = (r8 > c8).astype(jnp.float32)
    ra = jax.lax.broadcasted_iota(jnp.int32, (128, 128), 0)
    cb = jax.lax.broadcasted_iota(jnp.int32, (128, 128), 1)
    tin128 = (ra <= cb).astype(jnp.float32)

    def groups_body(i, counters):
        c_t = combo_s[i]
        rank_acc = jnp.zeros((8, 128), jnp.float32)
        new_counters = []
        for c in range(64):
            mask = (c_t == jnp.int32(c)).astype(jnp.float32)
            if c in VALID_IDS:
                lane_cs = jax.lax.dot_general(
                    mask, tin128, (((1,), (0,)), ((), ())),
                    preferred_element_type=jnp.float32)
                sub_tot = lane_cs[:, 127:128]
                excl_sub = jax.lax.dot_general(
                    ltri8, sub_tot, (((1,), (0,)), ((), ())),
                    preferred_element_type=jnp.float32)
                prefix_excl = lane_cs - mask + excl_sub
                rank_acc = rank_acc + mask * (prefix_excl + counters[c])
            new_counters.append(counters[c] + jnp.sum(mask))
        rank_s[i] = rank_acc
        return tuple(new_counters)

    counts = jax.lax.fori_loop(
        0, NBLK, groups_body, tuple(jnp.float32(0.0) for _ in range(64)))

    present = [(counts[c] > 0.0).astype(jnp.int32) for c in range(64)]
    pref = []
    acc = izero
    for c in range(64):
        acc = acc + present[c]
        pref.append(acc)
    combo_keys = {}
    for c in VALID_IDS:
        idx = (pref[c] - 1).astype(U32)
        k0c, k1c = _threefry2x32(U32(0), U32(42), U32(0), idx)
        combo_keys[c] = (k0c, k1c)

    sqrt2 = jnp.float32(np.sqrt(np.float32(2.0)))
    inv_ln2 = jnp.float32(1.0 / np.log(2.0))

    def main_body(i, carry):
        acc_pen, acc_log = carry
        c_t = combo_s[i]
        n_t = c_t // 16
        k_t = (c_t // 4) % 4
        m_t = c_t % 4
        valid = (m_t < n_t) & (k_t >= 1) & (k_t <= n_t)

        k0r = jnp.zeros((8, 128), U32)
        k1r = jnp.zeros((8, 128), U32)
        for c in VALID_IDS:
            sel = c_t == c
            k0c, k1c = combo_keys[c]
            k0r = jnp.where(sel, k0c, k0r)
            k1r = jnp.where(sel, k1c, k1r)

        rank_t = rank_s[i].astype(jnp.int32)
        base = (rank_t * (S * k_t)).astype(U32)
        ku = k_t.astype(U32)

        pm = [[None] * 2 for _ in range(2)]
        pmb = [[None] * 2 for _ in range(2)]
        for j in range(2):
            for jj in range(2):
                keep = (j < k_t) & (jj < (n_t - k_t))
                pv = jnp.where(keep, p_ref[j * 3 + jj, i], jnp.float32(0.0))
                pm[j][jj] = pv
                pmb[j][jj] = pv.astype(jnp.bfloat16).astype(jnp.float32)

        k_is1 = k_t == 1
        neg1 = jnp.float32(-1.0)
        max_hm = jnp.full((8, 128), -jnp.inf, jnp.float32)
        for s in range(S):
            a = []
            abf = []
            for j in range(3):
                cnt = base + U32(s) * ku + U32(j)
                o0, o1 = _threefry2x32(k0r, k1r, U32(0), cnt)
                aj = _normal_from_bits(o0 ^ o1)
                aj = jnp.where(j < k_t, aj, jnp.float32(0.0))
                a.append(aj)
                abf.append(aj.astype(jnp.bfloat16).astype(jnp.float32))
            mags = []
            for j in range(3):
                ident = jnp.where(k_is1, a[j], abf[j])
                mags.append(jnp.where(j < k_t, jnp.abs(ident), neg1))
            for jj in range(2):
                e1 = a[0] * pm[0][jj]
                e2 = abf[0] * pmb[0][jj] + abf[1] * pmb[1][jj]
                e = jnp.where(k_is1, e1, e2)
                mags.append(jnp.where(jj < (n_t - k_t), jnp.abs(e), neg1))
            v0 = mags[0]
            v1 = jnp.where(k_t >= 2, mags[1], mags[3])
            v2 = jnp.where(k_t == 3, mags[2],
                           jnp.where(k_t == 2, mags[3], mags[4]))
            hi01 = jnp.maximum(v0, v1)
            lo01 = jnp.minimum(v0, v1)
            t1m = jnp.maximum(lo01, v2)
            t2 = jnp.minimum(lo01, v2)
            t0 = jnp.maximum(hi01, t1m)
            t1 = jnp.minimum(hi01, t1m)
            t_m = jnp.where(m_t == 0, t0,
                            jnp.where(m_t == 1, t1, t2))
            hm = t0 / (t_m + EPS)
            max_hm = jnp.maximum(max_hm, hm)

        yp = yp_ref[i]
        pen = jnp.where(valid, jnp.maximum(max_hm - yp, jnp.float32(0.0)),
                        jnp.float32(0.0))

        ypc = jnp.maximum(yp, EPS)
        ytc = jnp.maximum(yt_ref[i], EPS)
        d = (jnp.log(ytc) - jnp.log(ypc)) * inv_ln2
        return acc_pen + jnp.sum(pen), acc_log + jnp.sum(d * d)

    acc_pen, acc_log = jax.lax.fori_loop(
        0, NBLK, main_body, (jnp.float32(0.0), jnp.float32(0.0)))

    logmse = acc_log / jnp.float32(B)
    violation = acc_pen / jnp.float32(B)
    out_logmse[0, 0] = logmse
    out_viol[0, 0] = violation
    out_total[0, 0] = logmse + LAMBDA_VIOLATION * violation


@functools.partial(jax.jit, static_argnames=("interpret",))
def kernel(y_pred, y_true, P_padded, params, interpret=False):
    yp = y_pred.reshape(NBLK, 8, 128)
    yt = y_true.reshape(NBLK, 8, 128)
    pc = jnp.transpose(P_padded[:, :3, :3], (1, 2, 0)).reshape(9, NBLK, 8, 128)
    pa = jnp.transpose(params.astype(jnp.int32), (1, 0)).reshape(3, NBLK, 8, 128)

    scal = jax.ShapeDtypeStruct((1, 1), jnp.float32)
    out_specs = [pl.BlockSpec(memory_space=pltpu.SMEM)] * 3
    total, logmse, violation = pl.pallas_call(
        _loss_kernel,
        out_shape=[scal, scal, scal],
        out_specs=out_specs,
        scratch_shapes=[
            pltpu.VMEM((NBLK, 8, 128), jnp.int32),
            pltpu.VMEM((NBLK, 8, 128), jnp.float32),
        ],
        interpret=interpret,
    )(yp, yt, pc, pa)
    return (total[0, 0], logmse[0, 0], violation[0, 0])
